# baseline (speedup 1.0000x reference)
import jax
import jax.numpy as jnp
from jax.experimental import pallas as pl

HEAD_NUM = 4


def _ident(x_ref, o_ref):
    o_ref[...] = x_ref[...]


def kernel(x, edge_index, Ws, Wb, As, Ab):
    N = x.shape[0]
    v_src = edge_index[0]
    v_tgt = edge_index[1]
    x_tildes = []
    for i in range(HEAD_NUM):
        Wx = x @ Ws[i] + Wb[i]
        e = jax.nn.leaky_relu(
            Wx[v_tgt] @ As[i][:128] + Wx[v_src] @ As[i][128:] + Ab[i],
            negative_slope=0.2,
        )
        m = jax.ops.segment_max(e, v_tgt, num_segments=N)
        ex = jnp.exp(e - m[v_tgt])
        s = jax.ops.segment_sum(ex, v_tgt, num_segments=N)
        alpha = ex / s[v_tgt]
        x_tildes.append(jax.ops.segment_sum(alpha[:, None] * Wx[v_src], v_tgt, num_segments=N))
    out = jnp.stack(x_tildes).mean(axis=0)
    return pl.pallas_call(
        _ident, out_shape=jax.ShapeDtypeStruct(out.shape, out.dtype)
    )(out)


# speedup vs baseline: 2.8982x; 2.8982x over previous
import functools

import jax
import jax.numpy as jnp
from jax import lax
from jax.experimental import pallas as pl
from jax.experimental.pallas import tpu as pltpu
from jax.experimental.pallas import tpu_sc as plsc

H = 4
D = 128
N = 10000
E = 320000

NW = 10048
NPAD = 10240
ROWS = 2560
EPAD = ROWS * 128
RPT = ROWS // 16
RBLKP = 2512
RBLK = 2048


def _project_body(x_ref, w_ref, b_ref, a_ref, ab_ref, wx_ref, alar_ref):
    wx = jnp.dot(x_ref[...], w_ref[...], preferred_element_type=jnp.float32)
    wx = wx + b_ref[...]
    for h in range(H):
        wx_ref[h] = wx[:, h * D:(h + 1) * D]
    alar = jnp.dot(wx, a_ref[...], preferred_element_type=jnp.float32)
    alar = alar + ab_ref[...]
    for h in range(H):
        alar_ref[h] = alar[:, 2 * h:2 * h + 2]


def _project(x_pad, Wcat, bcat, Amat, ab8):
    return pl.pallas_call(
        _project_body,
        grid=(NW // RBLKP,),
        in_specs=[
            pl.BlockSpec((RBLKP, D), lambda i: (i, 0)),
            pl.BlockSpec((D, H * D), lambda i: (0, 0)),
            pl.BlockSpec((1, H * D), lambda i: (0, 0)),
            pl.BlockSpec((H * D, 2 * H), lambda i: (0, 0)),
            pl.BlockSpec((1, 2 * H), lambda i: (0, 0)),
        ],
        out_specs=[
            pl.BlockSpec((H, RBLKP, D), lambda i: (0, i, 0)),
            pl.BlockSpec((H, RBLKP, 2), lambda i: (0, i, 0)),
        ],
        out_shape=[
            jax.ShapeDtypeStruct((H, NW, D), jnp.float32),
            jax.ShapeDtypeStruct((H, NW, 2), jnp.float32),
        ],
    )(x_pad, Wcat, bcat, Amat, ab8)


_MESH = plsc.VectorSubcoreMesh(core_axis_name="c", subcore_axis_name="s")

NQ = 2048
NR = 5
NPO = NR * NQ
ACCR = NQ + 8
RPTE = RPT // 2


@functools.partial(
    pl.kernel,
    out_type=(
        jax.ShapeDtypeStruct((2, H, NPO, D), jnp.float32),
        jax.ShapeDtypeStruct((H, 1, 2 * NR * 16 * NQ), jnp.float32),
    ),
    mesh=_MESH,
    compiler_params=pltpu.CompilerParams(needs_layout_passes=False),
    scratch_types=[
        pltpu.VMEM((160, 128), jnp.float32),
        pltpu.VMEM((RPTE, 128), jnp.int32),
        pltpu.VMEM((RPTE, 128), jnp.int32),
        pltpu.VMEM((3, 128, D), jnp.float32),
        pltpu.VMEM((3, 128), jnp.int32),
        pltpu.VMEM((NQ + 16,), jnp.float32),
        pltpu.VMEM_SHARED((ACCR, D), jnp.float32),
        pltpu.SemaphoreType.DMA((3,)),
        pltpu.SemaphoreType.DMA((3,)),
    ],
)
def _sc_gat(wx_flat, alar_hbm, tgt_hbm, src_hbm, acc_hbm, s_hbm,
            alar_t, tgt_il, src_adj, rowbuf, remap, s_loc, acc_sp,
            gsem, ssem):
    c = lax.axis_index("c")
    sid = lax.axis_index("s")
    r0 = (c * 16 + sid) * RPTE
    pltpu.sync_copy(tgt_hbm.at[pl.ds(r0, RPTE)], tgt_il)
    pltpu.sync_copy(src_hbm.at[pl.ds(r0, RPTE)], src_adj)
    zv = jnp.zeros((16,), jnp.float32)

    def _zero_rowbuf0():
        def _zr(i, carry):
            for k in range(D // 16):
                rowbuf[0, i, pl.ds(k * 16, 16)] = zv
            return carry
        lax.fori_loop(0, 128, _zr, 0)

    def _pass(hq, carry0):
        h = hq // NR
        q = hq - NR * h
        hNW = h * NW
        hbase = q * NQ
        wxh = wx_flat.at[pl.ds(hNW, NW)]
        pltpu.sync_copy(alar_hbm.at[pl.ds(h * 160, 160)], alar_t)
        _zero_rowbuf0()
        zb = sid * 128
        pltpu.sync_copy(rowbuf.at[0], acc_sp.at[pl.ds(zb, 128)])

        def _zs(i, carry):
            s_loc[pl.ds(i * 16, 16)] = zv
            return carry

        lax.fori_loop(0, (NQ + 16) // 16, _zs, 0)
        plsc.subcore_barrier()

        for qq in range(2):
            pltpu.async_copy(wxh.at[src_adj.at[qq]], rowbuf.at[qq],
                             gsem.at[qq])

        def _body(j, carry):
            par = lax.rem(j, 3)
            pltpu.make_async_copy(wx_flat.at[pl.ds(0, 128)], rowbuf.at[par],
                                  gsem.at[par]).wait()

            def _scale(g, carry2):
                tvo = tgt_il[j, pl.ds(g * 16, 16)]
                svo = src_adj[j, pl.ds(g * 16, 16)]
                ti = tvo * 2
                si = svo * 2 + 1
                al = plsc.load_gather(alar_t, [ti >> 7, ti & 127])
                ar = plsc.load_gather(alar_t, [si >> 7, si & 127])
                e = al + ar
                ex = jnp.exp(jnp.maximum(e, 0.2 * e))
                rel = tvo - hbase
                inhalf = (rel >= 0) & (rel < NQ)
                idxp = jnp.where(inhalf, rel, NQ + (tvo & 7))
                plsc.addupdate_scatter(s_loc, [idxp], ex)
                remap[par, pl.ds(g * 16, 16)] = idxp
                for i in range(16):
                    b = jnp.full((16,), ex[i], jnp.float32)
                    for k in range(D // 16):
                        rowbuf[par, g * 16 + i, pl.ds(k * 16, 16)] = (
                            rowbuf[par, g * 16 + i, pl.ds(k * 16, 16)] * b)
                return carry2

            lax.fori_loop(0, 8, _scale, 0)
            pltpu.async_copy(rowbuf.at[par], acc_sp.at[remap.at[par]],
                             ssem.at[par], add=True)
            q2 = j + 2

            @pl.when(q2 < RPTE)
            def _next():
                qp = lax.rem(q2, 3)

                @pl.when(j >= 1)
                def _free():
                    pltpu.make_async_copy(rowbuf.at[qp],
                                          acc_sp.at[pl.ds(0, 128)],
                                          ssem.at[qp]).wait()

                pltpu.async_copy(wxh.at[src_adj.at[q2]], rowbuf.at[qp],
                                 gsem.at[qp])
            return carry

        lax.fori_loop(0, RPTE, _body, 0)
        for par in range(3):
            pltpu.make_async_copy(rowbuf.at[par], acc_sp.at[pl.ds(0, 128)],
                                  ssem.at[par]).wait()
        pltpu.sync_copy(
            s_loc.at[pl.ds(0, NQ)],
            s_hbm.at[h].at[0].at[pl.ds((((q * 2 + c) * 16) + sid) * NQ, NQ)])
        plsc.subcore_barrier()
        pltpu.sync_copy(acc_sp.at[pl.ds(zb, 128)],
                        acc_hbm.at[c].at[h].at[pl.ds(hbase + zb, 128)])
        return carry0

    lax.fori_loop(0, NR * H, _pass, 0)


def _finish_body(acc_ref, s_ref, o_ref):
    s_sum = jnp.sum(s_ref[...], axis=(1, 2, 3))
    res = jnp.zeros((RBLK, D), jnp.float32)
    for h in range(H):
        sh = s_sum[h]
        inv = jnp.where(sh > 0, 1.0 / jnp.where(sh > 0, sh, 1.0), 0.0)
        res = res + (acc_ref[0, h] + acc_ref[1, h]) * inv[:, None]
    o_ref[...] = res * 0.25


def _finish(acc, s_part):
    return pl.pallas_call(
        _finish_body,
        grid=(NPO // RBLK,),
        in_specs=[
            pl.BlockSpec((2, H, RBLK, D), lambda i: (0, 0, i, 0)),
            pl.BlockSpec((H, 1, 2, 16, NQ), lambda i: (0, i, 0, 0, 0)),
        ],
        out_specs=pl.BlockSpec((RBLK, D), lambda i: (i, 0)),
        out_shape=jax.ShapeDtypeStruct((NPO, D), jnp.float32),
    )(acc, s_part)


def kernel(x, edge_index, Ws, Wb, As, Ab):
    src = edge_index[0].astype(jnp.int32)
    tgt = edge_index[1].astype(jnp.int32)
    padv = N + jnp.arange(EPAD - E, dtype=jnp.int32) % 16
    srcp = jnp.concatenate([src, padv]).reshape(ROWS, 128)
    tgtp = jnp.concatenate([tgt, padv]).reshape(ROWS, 128)
    x_pad = jnp.pad(x, ((0, NW - N), (0, 0)))

    Wcat = jnp.transpose(Ws, (1, 0, 2)).reshape(D, H * D)
    bcat = Wb.reshape(1, H * D)
    Amat = jnp.zeros((H * D, 2 * H), jnp.float32)
    ab8 = jnp.zeros((1, 2 * H), jnp.float32)
    for h in range(H):
        Amat = Amat.at[h * D:(h + 1) * D, 2 * h].set(As[h, :D])
        Amat = Amat.at[h * D:(h + 1) * D, 2 * h + 1].set(As[h, D:])
        ab8 = ab8.at[0, 2 * h].set(Ab[h])

    wx4, alar = _project(x_pad, Wcat, bcat, Amat, ab8)
    alar = jnp.pad(alar.reshape(H, NW * 2), ((0, 0), (0, 20480 - NW * 2)))
    alar = alar.reshape(H * 160, 128)
    acc, s_part = _sc_gat(wx4.reshape(H * NW, D), alar, tgtp, srcp)
    out = _finish(acc, s_part.reshape(H, NR, 2, 16, NQ))
    return out[:N]


# speedup vs baseline: 2.9052x; 1.0024x over previous
import functools

import jax
import jax.numpy as jnp
from jax import lax
from jax.experimental import pallas as pl
from jax.experimental.pallas import tpu as pltpu
from jax.experimental.pallas import tpu_sc as plsc

H = 4
D = 128
N = 10000
E = 320000

NW = 10048
NPAD = 10240
ROWS = 2560
EPAD = ROWS * 128
RPT = ROWS // 16
RBLKP = 2512
RBLK = 2048


def _project_body(x_ref, w_ref, b_ref, a_ref, ab_ref, wx_ref, alar_ref):
    wx = jnp.dot(x_ref[...], w_ref[...], preferred_element_type=jnp.float32)
    wx = wx + b_ref[...]
    for h in range(H):
        wx_ref[h] = wx[:, h * D:(h + 1) * D]
    alar = jnp.dot(wx, a_ref[...], preferred_element_type=jnp.float32)
    alar = alar + ab_ref[...]
    for h in range(H):
        alar_ref[h] = alar[:, 2 * h:2 * h + 2]


def _project(x_pad, Wcat, bcat, Amat, ab8):
    return pl.pallas_call(
        _project_body,
        grid=(NW // RBLKP,),
        in_specs=[
            pl.BlockSpec((RBLKP, D), lambda i: (i, 0)),
            pl.BlockSpec((D, H * D), lambda i: (0, 0)),
            pl.BlockSpec((1, H * D), lambda i: (0, 0)),
            pl.BlockSpec((H * D, 2 * H), lambda i: (0, 0)),
            pl.BlockSpec((1, 2 * H), lambda i: (0, 0)),
        ],
        out_specs=[
            pl.BlockSpec((H, RBLKP, D), lambda i: (0, i, 0)),
            pl.BlockSpec((H, RBLKP, 2), lambda i: (0, i, 0)),
        ],
        out_shape=[
            jax.ShapeDtypeStruct((H, NW, D), jnp.float32),
            jax.ShapeDtypeStruct((H, NW, 2), jnp.float32),
        ],
    )(x_pad, Wcat, bcat, Amat, ab8)


_MESH = plsc.VectorSubcoreMesh(core_axis_name="c", subcore_axis_name="s")

NQ = 2048
NR = 5
NPO = NR * NQ
ACCR = NQ + 64
RPTE = RPT // 2


@functools.partial(
    pl.kernel,
    out_type=(
        jax.ShapeDtypeStruct((2, H, NPO, D), jnp.float32),
        jax.ShapeDtypeStruct((H, 1, 2 * NR * 16 * NQ), jnp.float32),
    ),
    mesh=_MESH,
    compiler_params=pltpu.CompilerParams(needs_layout_passes=False),
    scratch_types=[
        pltpu.VMEM((160, 128), jnp.float32),
        pltpu.VMEM((RPTE, 128), jnp.int32),
        pltpu.VMEM((RPTE, 128), jnp.int32),
        pltpu.VMEM((3, 128, D), jnp.float32),
        pltpu.VMEM((3, 128), jnp.int32),
        pltpu.VMEM((NQ + 16,), jnp.float32),
        pltpu.VMEM_SHARED((ACCR, D), jnp.float32),
        pltpu.SemaphoreType.DMA((3,)),
        pltpu.SemaphoreType.DMA((3,)),
    ],
)
def _sc_gat(wx_flat, alar_hbm, tgt_hbm, src_hbm, acc_hbm, s_hbm,
            alar_t, tgt_il, src_adj, rowbuf, remap, s_loc, acc_sp,
            gsem, ssem):
    c = lax.axis_index("c")
    sid = lax.axis_index("s")
    r0 = (c * 16 + sid) * RPTE
    pltpu.sync_copy(tgt_hbm.at[pl.ds(r0, RPTE)], tgt_il)
    pltpu.sync_copy(src_hbm.at[pl.ds(r0, RPTE)], src_adj)
    zv = jnp.zeros((16,), jnp.float32)

    def _zero_rowbuf0():
        def _zr(i, carry):
            for k in range(D // 16):
                rowbuf[0, i, pl.ds(k * 16, 16)] = zv
            return carry
        lax.fori_loop(0, 128, _zr, 0)

    def _pass(hq, carry0):
        h = hq // NR
        q = hq - NR * h
        hNW = h * NW
        hbase = q * NQ
        wxh = wx_flat.at[pl.ds(hNW, NW)]
        pltpu.sync_copy(alar_hbm.at[pl.ds(h * 160, 160)], alar_t)
        _zero_rowbuf0()
        zb = sid * 128
        pltpu.sync_copy(rowbuf.at[0], acc_sp.at[pl.ds(zb, 128)])

        def _zs(i, carry):
            s_loc[pl.ds(i * 16, 16)] = zv
            return carry

        lax.fori_loop(0, (NQ + 16) // 16, _zs, 0)
        plsc.subcore_barrier()

        for qq in range(2):
            pltpu.async_copy(wxh.at[src_adj.at[qq]], rowbuf.at[qq],
                             gsem.at[qq])

        def _body(j, carry):
            par = lax.rem(j, 3)
            pltpu.make_async_copy(wx_flat.at[pl.ds(0, 128)], rowbuf.at[par],
                                  gsem.at[par]).wait()

            def _scale(g, carry2):
                tvo = tgt_il[j, pl.ds(g * 16, 16)]
                svo = src_adj[j, pl.ds(g * 16, 16)]
                ti = tvo * 2
                si = svo * 2 + 1
                al = plsc.load_gather(alar_t, [ti >> 7, ti & 127])
                ar = plsc.load_gather(alar_t, [si >> 7, si & 127])
                e = al + ar
                ex = jnp.exp(jnp.maximum(e, 0.2 * e))
                rel = tvo - hbase
                inhalf = (rel >= 0) & (rel < NQ)
                idxp = jnp.where(inhalf, rel, NQ + (tvo & 63))
                plsc.addupdate_scatter(s_loc, [idxp], ex)
                remap[par, pl.ds(g * 16, 16)] = idxp
                for i in range(16):
                    b = jnp.full((16,), ex[i], jnp.float32)
                    for k in range(D // 16):
                        rowbuf[par, g * 16 + i, pl.ds(k * 16, 16)] = (
                            rowbuf[par, g * 16 + i, pl.ds(k * 16, 16)] * b)
                return carry2

            lax.fori_loop(0, 8, _scale, 0)
            pltpu.async_copy(rowbuf.at[par], acc_sp.at[remap.at[par]],
                             ssem.at[par], add=True)
            q2 = j + 2

            @pl.when(q2 < RPTE)
            def _next():
                qp = lax.rem(q2, 3)

                @pl.when(j >= 1)
                def _free():
                    pltpu.make_async_copy(rowbuf.at[qp],
                                          acc_sp.at[pl.ds(0, 128)],
                                          ssem.at[qp]).wait()

                pltpu.async_copy(wxh.at[src_adj.at[q2]], rowbuf.at[qp],
                                 gsem.at[qp])
            return carry

        lax.fori_loop(0, RPTE, _body, 0)
        for par in range(3):
            pltpu.make_async_copy(rowbuf.at[par], acc_sp.at[pl.ds(0, 128)],
                                  ssem.at[par]).wait()
        pltpu.sync_copy(
            s_loc.at[pl.ds(0, NQ)],
            s_hbm.at[h].at[0].at[pl.ds((((q * 2 + c) * 16) + sid) * NQ, NQ)])
        plsc.subcore_barrier()
        pltpu.sync_copy(acc_sp.at[pl.ds(zb, 128)],
                        acc_hbm.at[c].at[h].at[pl.ds(hbase + zb, 128)])
        return carry0

    lax.fori_loop(0, NR * H, _pass, 0)


def _finish_body(acc_ref, s_ref, o_ref):
    s_sum = jnp.sum(s_ref[...], axis=(1, 2, 3))
    res = jnp.zeros((RBLK, D), jnp.float32)
    for h in range(H):
        sh = s_sum[h]
        inv = jnp.where(sh > 0, 1.0 / jnp.where(sh > 0, sh, 1.0), 0.0)
        res = res + (acc_ref[0, h] + acc_ref[1, h]) * inv[:, None]
    o_ref[...] = res * 0.25


def _finish(acc, s_part):
    return pl.pallas_call(
        _finish_body,
        grid=(NPO // RBLK,),
        in_specs=[
            pl.BlockSpec((2, H, RBLK, D), lambda i: (0, 0, i, 0)),
            pl.BlockSpec((H, 1, 2, 16, NQ), lambda i: (0, i, 0, 0, 0)),
        ],
        out_specs=pl.BlockSpec((RBLK, D), lambda i: (i, 0)),
        out_shape=jax.ShapeDtypeStruct((NPO, D), jnp.float32),
    )(acc, s_part)


def kernel(x, edge_index, Ws, Wb, As, Ab):
    src = edge_index[0].astype(jnp.int32)
    tgt = edge_index[1].astype(jnp.int32)
    padv = N + jnp.arange(EPAD - E, dtype=jnp.int32) % 16
    srcp = jnp.concatenate([src, padv]).reshape(ROWS, 128)
    tgtp = jnp.concatenate([tgt, padv]).reshape(ROWS, 128)
    x_pad = jnp.pad(x, ((0, NW - N), (0, 0)))

    Wcat = jnp.transpose(Ws, (1, 0, 2)).reshape(D, H * D)
    bcat = Wb.reshape(1, H * D)
    Amat = jnp.zeros((H * D, 2 * H), jnp.float32)
    ab8 = jnp.zeros((1, 2 * H), jnp.float32)
    for h in range(H):
        Amat = Amat.at[h * D:(h + 1) * D, 2 * h].set(As[h, :D])
        Amat = Amat.at[h * D:(h + 1) * D, 2 * h + 1].set(As[h, D:])
        ab8 = ab8.at[0, 2 * h].set(Ab[h])

    wx4, alar = _project(x_pad, Wcat, bcat, Amat, ab8)
    alar = jnp.pad(alar.reshape(H, NW * 2), ((0, 0), (0, 20480 - NW * 2)))
    alar = alar.reshape(H * 160, 128)
    acc, s_part = _sc_gat(wx4.reshape(H * NW, D), alar, tgtp, srcp)
    out = _finish(acc, s_part.reshape(H, NR, 2, 16, NQ))
    return out[:N]


# speedup vs baseline: 4.3013x; 1.4806x over previous
import functools

import jax
import jax.numpy as jnp
from jax import lax
from jax.experimental import pallas as pl
from jax.experimental.pallas import tpu as pltpu
from jax.experimental.pallas import tpu_sc as plsc

H = 4
D = 128
N = 10000
E = 320000

NW = 10048
NPAD = 10240
ROWS = 2560
EPAD = ROWS * 128
RPT = ROWS // 16
RBLKP = 2512
RBLK = 2048


def _project_body(x_ref, w_ref, b_ref, a_ref, ab_ref, wx_ref, alar_ref):
    wx = jnp.dot(x_ref[...], w_ref[...], preferred_element_type=jnp.float32)
    wx = wx + b_ref[...]
    for h in range(H):
        wx_ref[h] = wx[:, h * D:(h + 1) * D]
    alar = jnp.dot(wx, a_ref[...], preferred_element_type=jnp.float32)
    alar = alar + ab_ref[...]
    for h in range(H):
        alar_ref[h] = alar[:, 2 * h:2 * h + 2]


def _project(x_pad, Wcat, bcat, Amat, ab8):
    return pl.pallas_call(
        _project_body,
        grid=(NW // RBLKP,),
        in_specs=[
            pl.BlockSpec((RBLKP, D), lambda i: (i, 0)),
            pl.BlockSpec((D, H * D), lambda i: (0, 0)),
            pl.BlockSpec((1, H * D), lambda i: (0, 0)),
            pl.BlockSpec((H * D, 2 * H), lambda i: (0, 0)),
            pl.BlockSpec((1, 2 * H), lambda i: (0, 0)),
        ],
        out_specs=[
            pl.BlockSpec((H, RBLKP, D), lambda i: (0, i, 0)),
            pl.BlockSpec((H, RBLKP, 2), lambda i: (0, i, 0)),
        ],
        out_shape=[
            jax.ShapeDtypeStruct((H, NW, D), jnp.float32),
            jax.ShapeDtypeStruct((H, NW, 2), jnp.float32),
        ],
    )(x_pad, Wcat, bcat, Amat, ab8)


_MESH = plsc.VectorSubcoreMesh(core_axis_name="c", subcore_axis_name="s")

NQ = 2048
NR = 5
NPO = NR * NQ
ACCR = NQ + 64
RPTE = RPT // 2


@functools.partial(
    pl.kernel,
    out_type=(
        jax.ShapeDtypeStruct((2, H, NPO, D), jnp.float32),
        jax.ShapeDtypeStruct((H, 1, 2 * NR * 16 * NQ), jnp.float32),
    ),
    mesh=_MESH,
    compiler_params=pltpu.CompilerParams(needs_layout_passes=False),
    scratch_types=[
        pltpu.VMEM((160, 128), jnp.float32),
        pltpu.VMEM((RPTE, 128), jnp.int32),
        pltpu.VMEM((RPTE, 128), jnp.int32),
        pltpu.VMEM((3, 128, D), jnp.float32),
        pltpu.VMEM((3, 128), jnp.int32),
        pltpu.VMEM((NQ + 16,), jnp.float32),
        pltpu.VMEM((16,), jnp.int32),
        pltpu.VMEM_SHARED((ACCR, D), jnp.float32),
        pltpu.SemaphoreType.DMA((3,)),
        pltpu.SemaphoreType.DMA((3,)),
    ],
)
def _sc_gat(wx_flat, alar_hbm, tgt_hbm, src_hbm, bounds_hbm, acc_hbm, s_hbm,
            alar_t, tgt_il, src_adj, rowbuf, remap, s_loc, bnd, acc_sp,
            gsem, ssem):
    c = lax.axis_index("c")
    sid = lax.axis_index("s")
    tix = c * 16 + sid
    r0 = tix * RPTE
    pltpu.sync_copy(tgt_hbm.at[pl.ds(r0, RPTE)], tgt_il)
    pltpu.sync_copy(src_hbm.at[pl.ds(r0, RPTE)], src_adj)
    pltpu.sync_copy(bounds_hbm.at[tix], bnd)
    zv = jnp.zeros((16,), jnp.float32)

    def _zero_rowbuf0():
        def _zr(i, carry):
            for k in range(D // 16):
                rowbuf[0, i, pl.ds(k * 16, 16)] = zv
            return carry
        lax.fori_loop(0, 128, _zr, 0)

    def _head(h, carry0):
        hNW = h * NW
        wxh = wx_flat.at[pl.ds(hNW, NW)]
        pltpu.sync_copy(alar_hbm.at[pl.ds(h * 160, 160)], alar_t)
        for q in range(NR):
            _pass(h, q, wxh)
        return carry0

    def _pass(h, q, wxh):
        hbase = q * NQ
        _zero_rowbuf0()
        zb = sid * 128
        pltpu.sync_copy(rowbuf.at[0], acc_sp.at[pl.ds(zb, 128)])

        def _zs(i, carry):
            s_loc[pl.ds(i * 16, 16)] = zv
            return carry

        lax.fori_loop(0, (NQ + 16) // 16, _zs, 0)
        plsc.subcore_barrier()
        bnd16 = bnd[pl.ds(0, 16)]
        a = bnd16[2 * q]
        b = bnd16[2 * q + 1]
        nrows = b - a

        @pl.when(nrows >= 1)
        def _p0():
            pltpu.async_copy(wxh.at[src_adj.at[a]], rowbuf.at[lax.rem(a, 3)],
                             gsem.at[lax.rem(a, 3)])

        @pl.when(nrows >= 2)
        def _p1():
            pltpu.async_copy(wxh.at[src_adj.at[a + 1]],
                             rowbuf.at[lax.rem(a + 1, 3)],
                             gsem.at[lax.rem(a + 1, 3)])

        def _body(j, carry):
            par = lax.rem(j, 3)
            pltpu.make_async_copy(wx_flat.at[pl.ds(0, 128)], rowbuf.at[par],
                                  gsem.at[par]).wait()

            def _scale(g, carry2):
                tvo = tgt_il[j, pl.ds(g * 16, 16)]
                svo = src_adj[j, pl.ds(g * 16, 16)]
                ti = tvo * 2
                si = svo * 2 + 1
                al = plsc.load_gather(alar_t, [ti >> 7, ti & 127])
                ar = plsc.load_gather(alar_t, [si >> 7, si & 127])
                e = al + ar
                ex = jnp.exp(jnp.maximum(e, 0.2 * e))
                rel = tvo - hbase
                inhalf = (rel >= 0) & (rel < NQ)
                idxp = jnp.where(inhalf, rel, NQ + (tvo & 63))
                plsc.addupdate_scatter(s_loc, [idxp], ex)
                remap[par, pl.ds(g * 16, 16)] = idxp
                for i in range(16):
                    b = jnp.full((16,), ex[i], jnp.float32)
                    for k in range(D // 16):
                        rowbuf[par, g * 16 + i, pl.ds(k * 16, 16)] = (
                            rowbuf[par, g * 16 + i, pl.ds(k * 16, 16)] * b)
                return carry2

            lax.fori_loop(0, 8, _scale, 0)
            pltpu.async_copy(rowbuf.at[par], acc_sp.at[remap.at[par]],
                             ssem.at[par], add=True)
            q2 = j + 2

            @pl.when(q2 < b)
            def _next():
                qp = lax.rem(q2, 3)

                @pl.when(j >= a + 1)
                def _free():
                    pltpu.make_async_copy(rowbuf.at[qp],
                                          acc_sp.at[pl.ds(0, 128)],
                                          ssem.at[qp]).wait()

                pltpu.async_copy(wxh.at[src_adj.at[q2]], rowbuf.at[qp],
                                 gsem.at[qp])
            return carry

        lax.fori_loop(a, b, _body, 0)
        for k in range(3):
            @pl.when(nrows > k)
            def _drain(k=k):
                pltpu.make_async_copy(rowbuf.at[lax.rem(b - 1 - k, 3)],
                                      acc_sp.at[pl.ds(0, 128)],
                                      ssem.at[lax.rem(b - 1 - k, 3)]).wait()
        pltpu.sync_copy(
            s_loc.at[pl.ds(0, NQ)],
            s_hbm.at[h].at[0].at[pl.ds((((q * 2 + c) * 16) + sid) * NQ, NQ)])
        plsc.subcore_barrier()
        pltpu.sync_copy(acc_sp.at[pl.ds(zb, 128)],
                        acc_hbm.at[c].at[h].at[pl.ds(hbase + zb, 128)])

    lax.fori_loop(0, H, _head, 0)


def _finish_body(acc_ref, s_ref, o_ref):
    s_sum = jnp.sum(s_ref[...], axis=(1, 2, 3))
    res = jnp.zeros((RBLK, D), jnp.float32)
    for h in range(H):
        sh = s_sum[h]
        inv = jnp.where(sh > 0, 1.0 / jnp.where(sh > 0, sh, 1.0), 0.0)
        res = res + (acc_ref[0, h] + acc_ref[1, h]) * inv[:, None]
    o_ref[...] = res * 0.25


def _finish(acc, s_part):
    return pl.pallas_call(
        _finish_body,
        grid=(NPO // RBLK,),
        in_specs=[
            pl.BlockSpec((2, H, RBLK, D), lambda i: (0, 0, i, 0)),
            pl.BlockSpec((H, 1, 2, 16, NQ), lambda i: (0, i, 0, 0, 0)),
        ],
        out_specs=pl.BlockSpec((RBLK, D), lambda i: (i, 0)),
        out_shape=jax.ShapeDtypeStruct((NPO, D), jnp.float32),
    )(acc, s_part)


def kernel(x, edge_index, Ws, Wb, As, Ab):
    src = edge_index[0].astype(jnp.int32)
    tgt = edge_index[1].astype(jnp.int32)
    key = tgt // NQ
    order = jnp.argsort(key, stable=True)
    src = src[order]
    tgt = tgt[order]
    padv = N + jnp.arange(EPAD - E, dtype=jnp.int32) % 16
    srcp = jnp.concatenate([src, padv]).reshape(ROWS, 128)
    tgtp = jnp.concatenate([tgt, padv]).reshape(ROWS, 128)
    counts = jnp.bincount(key, length=NR).at[NR - 1].add(EPAD - E)
    estart = jnp.concatenate([jnp.zeros((1,), jnp.int32),
                              jnp.cumsum(counts).astype(jnp.int32)])
    t0 = jnp.arange(32, dtype=jnp.int32)[:, None] * (RPTE * 128)
    lo = jnp.clip(estart[None, :-1] // 128 - t0 // 128, 0, RPTE)
    hi = jnp.clip(-(-estart[None, 1:] // 128) - t0 // 128, 0, RPTE)
    bounds = jnp.stack([lo, hi], axis=-1).astype(jnp.int32)
    bounds = bounds.reshape(32, NR * 2)
    bounds = jnp.pad(bounds, ((0, 0), (0, 16 - NR * 2)))
    x_pad = jnp.pad(x, ((0, NW - N), (0, 0)))

    Wcat = jnp.transpose(Ws, (1, 0, 2)).reshape(D, H * D)
    bcat = Wb.reshape(1, H * D)
    Amat = jnp.zeros((H * D, 2 * H), jnp.float32)
    ab8 = jnp.zeros((1, 2 * H), jnp.float32)
    for h in range(H):
        Amat = Amat.at[h * D:(h + 1) * D, 2 * h].set(As[h, :D])
        Amat = Amat.at[h * D:(h + 1) * D, 2 * h + 1].set(As[h, D:])
        ab8 = ab8.at[0, 2 * h].set(Ab[h])

    wx4, alar = _project(x_pad, Wcat, bcat, Amat, ab8)
    alar = jnp.pad(alar.reshape(H, NW * 2), ((0, 0), (0, 20480 - NW * 2)))
    alar = alar.reshape(H * 160, 128)
    acc, s_part = _sc_gat(wx4.reshape(H * NW, D), alar, tgtp, srcp, bounds)
    out = _finish(acc, s_part.reshape(H, NR, 2, 16, NQ))
    return out[:N]


# speedup vs baseline: 4.3102x; 1.0021x over previous
import functools

import jax
import jax.numpy as jnp
from jax import lax
from jax.experimental import pallas as pl
from jax.experimental.pallas import tpu as pltpu
from jax.experimental.pallas import tpu_sc as plsc

H = 4
D = 128
N = 10000
E = 320000

NW = 10048
NPAD = 10240
ROWS = 2560
EPAD = ROWS * 128
RPT = ROWS // 16
RBLKP = 2512
RBLK = 2048


def _project_body(x_ref, w_ref, b_ref, a_ref, ab_ref, wx_ref, alar_ref):
    wx = jnp.dot(x_ref[...], w_ref[...], preferred_element_type=jnp.float32)
    wx = wx + b_ref[...]
    for h in range(H):
        wx_ref[h] = wx[:, h * D:(h + 1) * D]
    alar = jnp.dot(wx, a_ref[...], preferred_element_type=jnp.float32)
    alar = alar + ab_ref[...]
    for h in range(H):
        alar_ref[h] = alar[:, 2 * h:2 * h + 2]


def _project(x_pad, Wcat, bcat, Amat, ab8):
    return pl.pallas_call(
        _project_body,
        grid=(NW // RBLKP,),
        in_specs=[
            pl.BlockSpec((RBLKP, D), lambda i: (i, 0)),
            pl.BlockSpec((D, H * D), lambda i: (0, 0)),
            pl.BlockSpec((1, H * D), lambda i: (0, 0)),
            pl.BlockSpec((H * D, 2 * H), lambda i: (0, 0)),
            pl.BlockSpec((1, 2 * H), lambda i: (0, 0)),
        ],
        out_specs=[
            pl.BlockSpec((H, RBLKP, D), lambda i: (0, i, 0)),
            pl.BlockSpec((H, RBLKP, 2), lambda i: (0, i, 0)),
        ],
        out_shape=[
            jax.ShapeDtypeStruct((H, NW, D), jnp.float32),
            jax.ShapeDtypeStruct((H, NW, 2), jnp.float32),
        ],
    )(x_pad, Wcat, bcat, Amat, ab8)


_MESH = plsc.VectorSubcoreMesh(core_axis_name="c", subcore_axis_name="s")

NQ = 2048
NR = 5
NPO = NR * NQ
ACCR = NQ + 64
RPTE = RPT // 2


@functools.partial(
    pl.kernel,
    out_type=(
        jax.ShapeDtypeStruct((2, H, NPO, D), jnp.float32),
        jax.ShapeDtypeStruct((H, 1, 2 * NR * 16 * NQ), jnp.float32),
    ),
    mesh=_MESH,
    compiler_params=pltpu.CompilerParams(needs_layout_passes=False),
    scratch_types=[
        pltpu.VMEM((160, 128), jnp.float32),
        pltpu.VMEM((RPTE, 128), jnp.int32),
        pltpu.VMEM((RPTE, 128), jnp.int32),
        pltpu.VMEM((3, 128, D), jnp.float32),
        pltpu.VMEM((3, 128), jnp.int32),
        pltpu.VMEM((NQ + 16,), jnp.float32),
        pltpu.VMEM((16,), jnp.int32),
        pltpu.VMEM((128, D), jnp.float32),
        pltpu.VMEM_SHARED((ACCR, D), jnp.float32),
        pltpu.SemaphoreType.DMA((3,)),
        pltpu.SemaphoreType.DMA((3,)),
    ],
)
def _sc_gat(wx_flat, alar_hbm, tgt_hbm, src_hbm, bounds_hbm, acc_hbm, s_hbm,
            alar_t, tgt_il, src_adj, rowbuf, remap, s_loc, bnd, zbuf, acc_sp,
            gsem, ssem):
    c = lax.axis_index("c")
    sid = lax.axis_index("s")
    tix = c * 16 + sid
    r0 = tix * RPTE
    pltpu.sync_copy(tgt_hbm.at[pl.ds(r0, RPTE)], tgt_il)
    pltpu.sync_copy(src_hbm.at[pl.ds(r0, RPTE)], src_adj)
    pltpu.sync_copy(bounds_hbm.at[tix], bnd)
    zv = jnp.zeros((16,), jnp.float32)

    def _zr(i, carry):
        for k in range(D // 16):
            zbuf[i, pl.ds(k * 16, 16)] = zv
        return carry

    lax.fori_loop(0, 128, _zr, 0)

    def _head(h, carry0):
        hNW = h * NW
        wxh = wx_flat.at[pl.ds(hNW, NW)]
        pltpu.sync_copy(alar_hbm.at[pl.ds(h * 160, 160)], alar_t)
        for q in range(NR):
            _pass(h, q, wxh)
        return carry0

    def _pass(h, q, wxh):
        hbase = q * NQ
        zb = sid * 128
        pltpu.sync_copy(zbuf, acc_sp.at[pl.ds(zb, 128)])

        def _zs(i, carry):
            s_loc[pl.ds(i * 16, 16)] = zv
            return carry

        lax.fori_loop(0, (NQ + 16) // 16, _zs, 0)
        plsc.subcore_barrier()
        bnd16 = bnd[pl.ds(0, 16)]
        a = bnd16[2 * q]
        b = bnd16[2 * q + 1]
        nrows = b - a

        @pl.when(nrows >= 1)
        def _p0():
            pltpu.async_copy(wxh.at[src_adj.at[a]], rowbuf.at[lax.rem(a, 3)],
                             gsem.at[lax.rem(a, 3)])

        @pl.when(nrows >= 2)
        def _p1():
            pltpu.async_copy(wxh.at[src_adj.at[a + 1]],
                             rowbuf.at[lax.rem(a + 1, 3)],
                             gsem.at[lax.rem(a + 1, 3)])

        def _body(j, carry):
            par = lax.rem(j, 3)
            pltpu.make_async_copy(wx_flat.at[pl.ds(0, 128)], rowbuf.at[par],
                                  gsem.at[par]).wait()

            def _scale(g, carry2):
                tvo = tgt_il[j, pl.ds(g * 16, 16)]
                svo = src_adj[j, pl.ds(g * 16, 16)]
                ti = tvo * 2
                si = svo * 2 + 1
                al = plsc.load_gather(alar_t, [ti >> 7, ti & 127])
                ar = plsc.load_gather(alar_t, [si >> 7, si & 127])
                e = al + ar
                ex = jnp.exp(jnp.maximum(e, 0.2 * e))
                rel = tvo - hbase
                inhalf = (rel >= 0) & (rel < NQ)
                idxp = jnp.where(inhalf, rel, NQ + (tvo & 63))
                plsc.addupdate_scatter(s_loc, [idxp], ex)
                remap[par, pl.ds(g * 16, 16)] = idxp
                for i in range(16):
                    b = jnp.full((16,), ex[i], jnp.float32)
                    rr = rowbuf.at[par, g * 16 + i]
                    for k in range(D // 16):
                        rr[pl.ds(k * 16, 16)] = rr[pl.ds(k * 16, 16)] * b
                return carry2

            lax.fori_loop(0, 8, _scale, 0)
            pltpu.async_copy(rowbuf.at[par], acc_sp.at[remap.at[par]],
                             ssem.at[par], add=True)
            q2 = j + 2

            @pl.when(q2 < b)
            def _next():
                qp = lax.rem(q2, 3)

                @pl.when(j >= a + 1)
                def _free():
                    pltpu.make_async_copy(rowbuf.at[qp],
                                          acc_sp.at[pl.ds(0, 128)],
                                          ssem.at[qp]).wait()

                pltpu.async_copy(wxh.at[src_adj.at[q2]], rowbuf.at[qp],
                                 gsem.at[qp])
            return carry

        lax.fori_loop(a, b, _body, 0)
        for k in range(3):
            @pl.when(nrows > k)
            def _drain(k=k):
                pltpu.make_async_copy(rowbuf.at[lax.rem(b - 1 - k, 3)],
                                      acc_sp.at[pl.ds(0, 128)],
                                      ssem.at[lax.rem(b - 1 - k, 3)]).wait()
        pltpu.sync_copy(
            s_loc.at[pl.ds(0, NQ)],
            s_hbm.at[h].at[0].at[pl.ds((((q * 2 + c) * 16) + sid) * NQ, NQ)])
        plsc.subcore_barrier()
        pltpu.sync_copy(acc_sp.at[pl.ds(zb, 128)],
                        acc_hbm.at[c].at[h].at[pl.ds(hbase + zb, 128)])

    lax.fori_loop(0, H, _head, 0)


def _finish_body(acc_ref, s_ref, o_ref):
    s_sum = jnp.sum(s_ref[...], axis=(1, 2, 3))
    res = jnp.zeros((RBLK, D), jnp.float32)
    for h in range(H):
        sh = s_sum[h]
        inv = jnp.where(sh > 0, 1.0 / jnp.where(sh > 0, sh, 1.0), 0.0)
        res = res + (acc_ref[0, h] + acc_ref[1, h]) * inv[:, None]
    o_ref[...] = res * 0.25


def _finish(acc, s_part):
    return pl.pallas_call(
        _finish_body,
        grid=(NPO // RBLK,),
        in_specs=[
            pl.BlockSpec((2, H, RBLK, D), lambda i: (0, 0, i, 0)),
            pl.BlockSpec((H, 1, 2, 16, NQ), lambda i: (0, i, 0, 0, 0)),
        ],
        out_specs=pl.BlockSpec((RBLK, D), lambda i: (i, 0)),
        out_shape=jax.ShapeDtypeStruct((NPO, D), jnp.float32),
    )(acc, s_part)


def kernel(x, edge_index, Ws, Wb, As, Ab):
    src = edge_index[0].astype(jnp.int32)
    tgt = edge_index[1].astype(jnp.int32)
    key = tgt // NQ
    order = jnp.argsort(key, stable=True)
    src = src[order]
    tgt = tgt[order]
    padv = N + jnp.arange(EPAD - E, dtype=jnp.int32) % 16
    srcp = jnp.concatenate([src, padv]).reshape(ROWS, 128)
    tgtp = jnp.concatenate([tgt, padv]).reshape(ROWS, 128)
    counts = jnp.bincount(key, length=NR).at[NR - 1].add(EPAD - E)
    estart = jnp.concatenate([jnp.zeros((1,), jnp.int32),
                              jnp.cumsum(counts).astype(jnp.int32)])
    t0 = jnp.arange(32, dtype=jnp.int32)[:, None] * (RPTE * 128)
    lo = jnp.clip(estart[None, :-1] // 128 - t0 // 128, 0, RPTE)
    hi = jnp.clip(-(-estart[None, 1:] // 128) - t0 // 128, 0, RPTE)
    bounds = jnp.stack([lo, hi], axis=-1).astype(jnp.int32)
    bounds = bounds.reshape(32, NR * 2)
    bounds = jnp.pad(bounds, ((0, 0), (0, 16 - NR * 2)))
    x_pad = jnp.pad(x, ((0, NW - N), (0, 0)))

    Wcat = jnp.transpose(Ws, (1, 0, 2)).reshape(D, H * D)
    bcat = Wb.reshape(1, H * D)
    Amat = jnp.zeros((H * D, 2 * H), jnp.float32)
    ab8 = jnp.zeros((1, 2 * H), jnp.float32)
    for h in range(H):
        Amat = Amat.at[h * D:(h + 1) * D, 2 * h].set(As[h, :D])
        Amat = Amat.at[h * D:(h + 1) * D, 2 * h + 1].set(As[h, D:])
        ab8 = ab8.at[0, 2 * h].set(Ab[h])

    wx4, alar = _project(x_pad, Wcat, bcat, Amat, ab8)
    alar = jnp.pad(alar.reshape(H, NW * 2), ((0, 0), (0, 20480 - NW * 2)))
    alar = alar.reshape(H * 160, 128)
    acc, s_part = _sc_gat(wx4.reshape(H * NW, D), alar, tgtp, srcp, bounds)
    out = _finish(acc, s_part.reshape(H, NR, 2, 16, NQ))
    return out[:N]
